# baseline (speedup 1.0000x reference)
import jax
import jax.numpy as jnp
from jax.experimental import pallas as pl
from jax.experimental.pallas import tpu as pltpu

N_PRED = 20000
M_GT = 1000
BN = 5120
N_PAD = 20480
M_PAD = 1024
NSTEPS = N_PAD // BN
SC_ROWS = 160


def _body(pred_ref, gtT_ref, sc_ref, prob_ref, iou_ref, acc_ref):
    i = pl.program_id(0)

    @pl.when(i == 0)
    def _init():
        acc_ref[...] = jnp.zeros_like(acc_ref)
        prob_ref[0, 0] = jnp.max(sc_ref[...])

    pred = pred_ref[...]
    px0 = pred[:, 0:1]
    py0 = pred[:, 1:2]
    px1 = pred[:, 2:3]
    py1 = pred[:, 3:4]

    g = gtT_ref[...]
    gx0 = g[0:1, :]
    gy0 = g[1:2, :]
    gx1 = g[2:3, :]
    gy1 = g[3:4, :]
    mask = (gx0 + gy0 + gx1 + gy1) != 0.0
    gx0 = jnp.where(mask, gx0, -2.0)
    gy0 = jnp.where(mask, gy0, -2.0)
    gx1 = jnp.where(mask, gx1, -1.0)
    gy1 = jnp.where(mask, gy1, -1.0)

    iw = jnp.maximum(jnp.minimum(px1, gx1) - jnp.maximum(px0, gx0), 0.0)
    ih = jnp.maximum(jnp.minimum(py1, gy1) - jnp.maximum(py0, gy0), 0.0)
    inter = iw * ih
    ap = (px1 - px0) * (py1 - py0)
    ag = (gx1 - gx0) * (gy1 - gy0)
    t = inter / (ap + ag)
    acc_ref[...] = jnp.maximum(acc_ref[...],
                               jnp.max(t, axis=0, keepdims=True))

    @pl.when(i == NSTEPS - 1)
    def _fin():
        tm = jnp.max(acc_ref[...])
        iou_ref[0, 0] = tm / (1.0 - tm)


@jax.jit
def kernel(pred_boxes, scores, gt_boxes):
    n = pred_boxes.shape[0]
    m = gt_boxes.shape[0]
    pad_pred = jnp.broadcast_to(
        jnp.array([-4.0, -4.0, -3.0, -3.0], jnp.float32), (N_PAD - n, 4))
    pred_p = jnp.concatenate([pred_boxes, pad_pred], axis=0)
    gtT = jnp.zeros((8, M_PAD), jnp.float32).at[:4, :m].set(gt_boxes.T)
    sc_p = jnp.full((SC_ROWS * 128,), -jnp.inf, jnp.float32).at[:n].set(scores)
    sc_p = sc_p.reshape(SC_ROWS, 128)

    prob, iou = pl.pallas_call(
        _body,
        grid=(NSTEPS,),
        in_specs=[
            pl.BlockSpec((BN, 4), lambda i: (i, 0)),
            pl.BlockSpec((8, M_PAD), lambda i: (0, 0)),
            pl.BlockSpec((SC_ROWS, 128), lambda i: (0, 0)),
        ],
        out_specs=[
            pl.BlockSpec(memory_space=pltpu.SMEM),
            pl.BlockSpec(memory_space=pltpu.SMEM),
        ],
        out_shape=[
            jax.ShapeDtypeStruct((1, 1), jnp.float32),
            jax.ShapeDtypeStruct((1, 1), jnp.float32),
        ],
        scratch_shapes=[pltpu.VMEM((1, M_PAD), jnp.float32)],
        compiler_params=pltpu.CompilerParams(
            dimension_semantics=("arbitrary",)),
    )(pred_p, gtT, sc_p)
    return (prob[0, 0], iou[0, 0])
